# baseline (speedup 1.0000x reference)
import functools

import jax
import jax.numpy as jnp
from jax import lax
from jax.experimental import pallas as pl
from jax.experimental.pallas import tpu as pltpu
from jax.experimental.pallas import tpu_sc as plsc

_NC = 2
_NS = 16
_SB = 128


def kernel(x, pe):
    batch, seq_len = x.shape
    max_len, d_model = pe.shape
    flat = seq_len * d_model
    pe_flat = pe.reshape(1, flat)
    rows_per_core = batch // _NC
    blocks_per_tile = rows_per_core // (_NS * _SB)
    fill_rows = _SB // _NS

    mesh = plsc.VectorSubcoreMesh(core_axis_name="c", subcore_axis_name="s")

    @functools.partial(
        pl.kernel,
        mesh=mesh,
        out_type=jax.ShapeDtypeStruct((batch, flat), jnp.float32),
        scratch_types=[
            pltpu.MemorySpace.VMEM_SHARED((_SB, flat), jnp.float32),
            pltpu.SemaphoreType.DMA,
        ],
    )
    def sc_bcast(pe_hbm, out_hbm, shared, sem):
        cid = lax.axis_index("c")
        sid = lax.axis_index("s")
        for r in range(fill_rows):
            pltpu.sync_copy(pe_hbm.at[0], shared.at[sid * fill_rows + r])
        plsc.subcore_barrier()
        base = cid * rows_per_core + sid * (_SB * blocks_per_tile)
        for j in range(blocks_per_tile):
            pltpu.make_async_copy(
                shared, out_hbm.at[pl.ds(base + j * _SB, _SB)], sem
            ).start()
        for j in range(blocks_per_tile):
            pltpu.make_async_copy(
                shared, out_hbm.at[pl.ds(base + j * _SB, _SB)], sem
            ).wait()

    out = sc_bcast(pe_flat)
    return out.reshape(batch, seq_len, d_model)


# speedup vs baseline: 1.1584x; 1.1584x over previous
import functools

import jax
import jax.numpy as jnp
from jax import lax
from jax.experimental import pallas as pl
from jax.experimental.pallas import tpu as pltpu
from jax.experimental.pallas import tpu_sc as plsc

_NC = 2
_NS = 16
_R = 4


def kernel(x, pe):
    batch, seq_len = x.shape
    max_len, d_model = pe.shape
    flat = seq_len * d_model
    pe_flat = pe.reshape(flat)
    nw = _NC * _NS
    rows_per_w = batch // nw
    n_chunks = rows_per_w // _R

    mesh = plsc.VectorSubcoreMesh(core_axis_name="c", subcore_axis_name="s")

    @functools.partial(
        pl.kernel,
        mesh=mesh,
        out_type=jax.ShapeDtypeStruct((batch, flat), jnp.float32),
        scratch_types=[
            pltpu.VMEM((_R, flat), jnp.float32),
            pltpu.SemaphoreType.DMA,
        ],
    )
    def sc_bcast(pe_hbm, out_hbm, buf, sem):
        wid = lax.axis_index("s") * _NC + lax.axis_index("c")
        base = wid * rows_per_w
        for r in range(_R):
            pltpu.sync_copy(pe_hbm, buf.at[r])
        for j in range(n_chunks):
            pltpu.make_async_copy(
                buf, out_hbm.at[pl.ds(base + j * _R, _R)], sem
            ).start()
        for j in range(n_chunks):
            pltpu.make_async_copy(
                buf, out_hbm.at[pl.ds(base + j * _R, _R)], sem
            ).wait()

    out = sc_bcast(pe_flat)
    return out.reshape(batch, seq_len, d_model)


# speedup vs baseline: 1.1769x; 1.0160x over previous
import functools

import jax
import jax.numpy as jnp
from jax import lax
from jax.experimental import pallas as pl
from jax.experimental.pallas import tpu as pltpu
from jax.experimental.pallas import tpu_sc as plsc

_NC = 2
_NS = 16
_R = 2


def kernel(x, pe):
    batch, seq_len = x.shape
    max_len, d_model = pe.shape
    flat = seq_len * d_model
    pe_flat = pe.reshape(flat)
    nw = _NC * _NS
    rows_per_w = batch // nw
    n_chunks = rows_per_w // _R

    mesh = plsc.VectorSubcoreMesh(core_axis_name="c", subcore_axis_name="s")

    @functools.partial(
        pl.kernel,
        mesh=mesh,
        out_type=jax.ShapeDtypeStruct((batch, flat), jnp.float32),
        scratch_types=[
            pltpu.VMEM((_R, flat), jnp.float32),
            pltpu.SemaphoreType.DMA,
        ],
    )
    def sc_bcast(pe_hbm, out_hbm, buf, sem):
        wid = lax.axis_index("s") * _NC + lax.axis_index("c")
        base = wid * rows_per_w
        for r in range(_R):
            pltpu.sync_copy(pe_hbm, buf.at[r])
        for j in range(n_chunks):
            pltpu.make_async_copy(
                buf, out_hbm.at[pl.ds(base + j * _R, _R)], sem
            ).start()
        for j in range(n_chunks):
            pltpu.make_async_copy(
                buf, out_hbm.at[pl.ds(base + j * _R, _R)], sem
            ).wait()

    out = sc_bcast(pe_flat)
    return out.reshape(batch, seq_len, d_model)


# speedup vs baseline: 1.1806x; 1.0031x over previous
import functools

import jax
import jax.numpy as jnp
from jax import lax
from jax.experimental import pallas as pl
from jax.experimental.pallas import tpu as pltpu
from jax.experimental.pallas import tpu_sc as plsc

_NC = 2
_NS = 16
_R = 1


def kernel(x, pe):
    batch, seq_len = x.shape
    max_len, d_model = pe.shape
    flat = seq_len * d_model
    pe_flat = pe.reshape(flat)
    nw = _NC * _NS
    rows_per_w = batch // nw
    n_chunks = rows_per_w // _R

    mesh = plsc.VectorSubcoreMesh(core_axis_name="c", subcore_axis_name="s")

    @functools.partial(
        pl.kernel,
        mesh=mesh,
        out_type=jax.ShapeDtypeStruct((batch, flat), jnp.float32),
        scratch_types=[
            pltpu.VMEM((_R, flat), jnp.float32),
            pltpu.SemaphoreType.DMA,
        ],
    )
    def sc_bcast(pe_hbm, out_hbm, buf, sem):
        wid = lax.axis_index("s") * _NC + lax.axis_index("c")
        base = wid * rows_per_w
        for r in range(_R):
            pltpu.sync_copy(pe_hbm, buf.at[r])
        for j in range(n_chunks):
            pltpu.make_async_copy(
                buf, out_hbm.at[pl.ds(base + j * _R, _R)], sem
            ).start()
        for j in range(n_chunks):
            pltpu.make_async_copy(
                buf, out_hbm.at[pl.ds(base + j * _R, _R)], sem
            ).wait()

    out = sc_bcast(pe_flat)
    return out.reshape(batch, seq_len, d_model)
